# baseline (speedup 1.0000x reference)
import jax
import jax.numpy as jnp
from jax.experimental import pallas as pl
from jax.experimental.pallas import tpu as pltpu

_IN = 4096
_OUT = 4096
_K = 1228
_BO = 512


def _do_select(x_ref, xm_ref):
    xv = x_ref[...]
    s = jnp.abs(xv)
    bits = jax.lax.bitcast_convert_type(s, jnp.int32)
    j16 = jax.lax.broadcasted_iota(jnp.int32, (16, 1), 0)

    t = jnp.int32(0)
    for p in range(8):
        shift = 28 - 4 * p
        cand = t + (j16 << shift)
        cnts = jnp.sum((bits >= cand).astype(jnp.int32), axis=1, keepdims=True)
        ok = (cnts >= _K) & (cand >= 0)
        d = jnp.sum(ok.astype(jnp.int32)) - 1
        t = t + (d << shift)

    gt = bits > t
    eq = bits == t
    r = _K - jnp.sum(gt.astype(jnp.int32))
    iota = jax.lax.broadcasted_iota(jnp.int32, (1, _IN), 1)
    eq_i = eq.astype(jnp.int32)

    pfx = jnp.int32(0)
    for p in range(3):
        shift = 8 - 4 * p
        cand = pfx + (j16 << shift)
        f = jnp.sum(jnp.where(iota < cand, eq_i, 0), axis=1, keepdims=True)
        d = jnp.maximum(jnp.sum((f < r).astype(jnp.int32)) - 1, 0)
        pfx = pfx + (d << shift)
    istar = jnp.where(r > 0, pfx + 1, 0)

    mask = gt | (eq & (iota < istar))
    xm_ref[...] = jnp.where(mask, xv, 0.0)


def _body2(x_ref, wa_ref, wb_ref, b_ref, oa_ref, ob_ref, xm_ref):
    g = pl.program_id(0)

    @pl.when(g == 0)
    def _select():
        _do_select(x_ref, xm_ref)

    xm = xm_ref[...]
    acc_a = jax.lax.dot_general(
        xm, wa_ref[...], (((1,), (1,)), ((), ())),
        preferred_element_type=jnp.float32,
    )
    acc_b = jax.lax.dot_general(
        xm, wb_ref[...], (((1,), (1,)), ((), ())),
        preferred_element_type=jnp.float32,
    )
    oa_ref[...] = acc_a + b_ref[0:1, :]
    ob_ref[...] = acc_b + b_ref[1:2, :]


@jax.jit
def _run(x2, w, b2):
    half = _OUT // 2
    nh = half // _BO
    wa = w[:half]
    wb = w[half:]
    bh = b2.reshape(2, half)
    oa, ob = pl.pallas_call(
        _body2,
        grid=(nh,),
        in_specs=[
            pl.BlockSpec((1, _IN), lambda g: (0, 0)),
            pl.BlockSpec((_BO, _IN), lambda g: (g, 0)),
            pl.BlockSpec((_BO, _IN), lambda g: (g, 0)),
            pl.BlockSpec((2, _BO), lambda g: (0, g)),
        ],
        out_specs=[
            pl.BlockSpec((1, _BO), lambda g: (0, g)),
            pl.BlockSpec((1, _BO), lambda g: (0, g)),
        ],
        out_shape=[
            jax.ShapeDtypeStruct((1, half), jnp.float32),
            jax.ShapeDtypeStruct((1, half), jnp.float32),
        ],
        scratch_shapes=[pltpu.VMEM((1, _IN), jnp.float32)],
    )(x2, wa, wb, bh)
    return jnp.concatenate([oa, ob], axis=1)


def kernel(x, weight, bias):
    bsz, seq, _ = x.shape
    out = _run(x.reshape(1, _IN), weight, bias.reshape(1, _OUT))
    return out.reshape(bsz, seq, _OUT)


# speedup vs baseline: 2.5540x; 2.5540x over previous
import jax
import jax.numpy as jnp
from jax.experimental import pallas as pl
from jax.experimental.pallas import tpu as pltpu

_IN = 4096
_OUT = 4096
_K = 1228
_BO = 512


def _do_select(x_ref, xm_ref):
    xv = x_ref[...]
    s = jnp.abs(xv)
    bits = jax.lax.bitcast_convert_type(s, jnp.int32)
    j16 = jax.lax.broadcasted_iota(jnp.int32, (16, 1), 0)

    t = jnp.int32(0)
    for p in range(8):
        shift = 28 - 4 * p
        cand = t + (j16 << shift)
        cnts = jnp.sum((bits >= cand).astype(jnp.int32), axis=1, keepdims=True)
        ok = (cnts >= _K) & (cand >= 0)
        d = jnp.sum(ok.astype(jnp.int32)) - 1
        t = t + (d << shift)

    gt = bits > t
    eq = bits == t
    r = _K - jnp.sum(gt.astype(jnp.int32))
    iota = jax.lax.broadcasted_iota(jnp.int32, (1, _IN), 1)
    eq_i = eq.astype(jnp.int32)

    pfx = jnp.int32(0)
    for p in range(3):
        shift = 8 - 4 * p
        cand = pfx + (j16 << shift)
        f = jnp.sum(jnp.where(iota < cand, eq_i, 0), axis=1, keepdims=True)
        d = jnp.maximum(jnp.sum((f < r).astype(jnp.int32)) - 1, 0)
        pfx = pfx + (d << shift)
    istar = jnp.where(r > 0, pfx + 1, 0)

    mask = gt | (eq & (iota < istar))
    xm_ref[...] = jnp.where(mask, xv, 0.0)


def _body2(x_ref, wa_ref, wb_ref, b_ref, oa_ref, ob_ref, xm_ref):
    g = pl.program_id(0)

    @pl.when(g == 0)
    def _select():
        _do_select(x_ref, xm_ref)

    xm = xm_ref[...]
    acc_a = jax.lax.dot_general(
        xm, wa_ref[...], (((1,), (1,)), ((), ())),
        preferred_element_type=jnp.float32,
    )
    acc_b = jax.lax.dot_general(
        xm, wb_ref[...], (((1,), (1,)), ((), ())),
        preferred_element_type=jnp.float32,
    )
    oa_ref[...] = acc_a + b_ref[0:1, :]
    ob_ref[...] = acc_b + b_ref[1:2, :]


@jax.jit
def _run(x2, w, b2):
    half = _OUT // 2
    nh = half // _BO
    bh = b2.reshape(2, half)
    oa, ob = pl.pallas_call(
        _body2,
        grid=(nh,),
        in_specs=[
            pl.BlockSpec((1, _IN), lambda g: (0, 0)),
            pl.BlockSpec((_BO, _IN), lambda g: (g, 0)),
            pl.BlockSpec((_BO, _IN), lambda g: (g + nh, 0)),
            pl.BlockSpec((2, _BO), lambda g: (0, g)),
        ],
        out_specs=[
            pl.BlockSpec((1, _BO), lambda g: (0, g)),
            pl.BlockSpec((1, _BO), lambda g: (0, g)),
        ],
        out_shape=[
            jax.ShapeDtypeStruct((1, half), jnp.float32),
            jax.ShapeDtypeStruct((1, half), jnp.float32),
        ],
        scratch_shapes=[pltpu.VMEM((1, _IN), jnp.float32)],
    )(x2, w, w, bh)
    return jnp.concatenate([oa, ob], axis=1)


def kernel(x, weight, bias):
    bsz, seq, _ = x.shape
    out = _run(x.reshape(1, _IN), weight, bias.reshape(1, _OUT))
    return out.reshape(bsz, seq, _OUT)


# speedup vs baseline: 2.8972x; 1.1343x over previous
import jax
import jax.numpy as jnp
from jax.experimental import pallas as pl
from jax.experimental.pallas import tpu as pltpu

_IN = 4096
_OUT = 4096
_K = 1228
_BO = 512
_NB = _OUT // _BO


def _do_select(x_ref, xm_ref):
    xv = x_ref[...]
    s = jnp.abs(xv)
    bits = jax.lax.bitcast_convert_type(s, jnp.int32)
    j16 = jax.lax.broadcasted_iota(jnp.int32, (16, 1), 0)

    t = jnp.int32(0)
    for p in range(8):
        shift = 28 - 4 * p
        cand = t + (j16 << shift)
        cnts = jnp.sum((bits >= cand).astype(jnp.int32), axis=1, keepdims=True)
        ok = (cnts >= _K) & (cand >= 0)
        d = jnp.sum(ok.astype(jnp.int32)) - 1
        t = t + (d << shift)

    gt = bits > t
    eq = bits == t
    r = _K - jnp.sum(gt.astype(jnp.int32))
    iota = jax.lax.broadcasted_iota(jnp.int32, (1, _IN), 1)
    eq_i = eq.astype(jnp.int32)

    pfx = jnp.int32(0)
    for p in range(3):
        shift = 8 - 4 * p
        cand = pfx + (j16 << shift)
        f = jnp.sum(jnp.where(iota < cand, eq_i, 0), axis=1, keepdims=True)
        d = jnp.maximum(jnp.sum((f < r).astype(jnp.int32)) - 1, 0)
        pfx = pfx + (d << shift)
    istar = jnp.where(r > 0, pfx + 1, 0)

    mask = gt | (eq & (iota < istar))
    xm_ref[...] = jnp.where(mask, xv, 0.0)


def _body(x_ref, w_ref, b_ref, o_ref, xm_ref):
    g = pl.program_id(0)

    @pl.when(g == 0)
    def _select():
        _do_select(x_ref, xm_ref)

    @pl.when(g > 0)
    def _mm():
        acc = jax.lax.dot_general(
            xm_ref[...], w_ref[...], (((1,), (1,)), ((), ())),
            preferred_element_type=jnp.float32,
        )
        o_ref[...] = acc + b_ref[...]


@jax.jit
def _run(x2, w, b2):
    def _wmap(g):
        return (jnp.maximum(g - 1, 0), 0)

    def _omap(g):
        return (0, jnp.maximum(g - 1, 0))

    return pl.pallas_call(
        _body,
        grid=(_NB + 1,),
        in_specs=[
            pl.BlockSpec((1, _IN), lambda g: (0, 0)),
            pl.BlockSpec((_BO, _IN), _wmap),
            pl.BlockSpec((1, _BO), _omap),
        ],
        out_specs=pl.BlockSpec((1, _BO), _omap),
        out_shape=jax.ShapeDtypeStruct((1, _OUT), jnp.float32),
        scratch_shapes=[pltpu.VMEM((1, _IN), jnp.float32)],
    )(x2, w, b2)


def kernel(x, weight, bias):
    bsz, seq, _ = x.shape
    out = _run(x.reshape(1, _IN), weight, bias.reshape(1, _OUT))
    return out.reshape(bsz, seq, _OUT)
